# baseline (speedup 1.0000x reference)
import functools

import jax
import jax.numpy as jnp
from jax import lax
from jax.experimental import pallas as pl
from jax.experimental.pallas import tpu as pltpu
from jax.experimental.pallas import tpu_sc as plsc

_N_E = 8192
_E_DIM = 32
_BETA = 0.2
_BM = 512
_GRID = _N_E // _BM

_NC = 2
_NS = 16
_NW = _NC * _NS
_PER_W = _N_E // _NW
_GCHUNK = 128


def _dist_body(z_ref, az_ref, embb_ref, emb_ref, ae_ref, aeh_ref,
               idx_ref, mind_ref, sm_ref, rs_ref, rq_ref):
    emb = emb_ref[...]

    zb = z_ref[...]
    az = az_ref[...]
    ae = ae_ref[...]
    p = lax.dot_general(zb, emb, (((1,), (1,)), ((), ())),
                        preferred_element_type=jnp.float32)
    d = (az + ae) - 2.0 * p
    mind = jnp.min(d, axis=1, keepdims=True)
    cols = lax.broadcasted_iota(jnp.int32, d.shape, 1)
    idx = jnp.min(jnp.where(d == mind, cols, _N_E), axis=1)
    idx_ref[0, 0, :] = idx
    mind_ref[0, 0, :] = mind[:, 0]

    i = pl.program_id(0)
    eb = embb_ref[...]
    aeh = aeh_ref[...]
    pc = lax.dot_general(eb, emb, (((1,), (1,)), ((), ())),
                         preferred_element_type=jnp.float32)
    u = aeh - pc
    rows = lax.broadcasted_iota(jnp.int32, u.shape, 0) + i * _BM
    um = jnp.where(rows == cols, jnp.float32(jnp.inf), u)
    sm_ref[0, 0, :] = jnp.min(um, axis=1)
    rs_ref[0, 0, :] = jnp.sum(u, axis=1)
    rq_ref[0, 0, :] = jnp.sum(u * u, axis=1)


_dist_call = pl.pallas_call(
    _dist_body,
    grid=(_GRID,),
    in_specs=[
        pl.BlockSpec((_BM, _E_DIM), lambda i: (i, 0)),
        pl.BlockSpec((_BM, 1), lambda i: (i, 0)),
        pl.BlockSpec((_BM, _E_DIM), lambda i: (i, 0)),
        pl.BlockSpec((_N_E, _E_DIM), lambda i: (0, 0)),
        pl.BlockSpec((1, _N_E), lambda i: (0, 0)),
        pl.BlockSpec((1, _N_E), lambda i: (0, 0)),
    ],
    out_specs=[
        pl.BlockSpec((1, 1, _BM), lambda i: (i, 0, 0)),
        pl.BlockSpec((1, 1, _BM), lambda i: (i, 0, 0)),
        pl.BlockSpec((1, 1, _BM), lambda i: (i, 0, 0)),
        pl.BlockSpec((1, 1, _BM), lambda i: (i, 0, 0)),
        pl.BlockSpec((1, 1, _BM), lambda i: (i, 0, 0)),
    ],
    out_shape=[
        jax.ShapeDtypeStruct((_GRID, 1, _BM), jnp.int32),
        jax.ShapeDtypeStruct((_GRID, 1, _BM), jnp.float32),
        jax.ShapeDtypeStruct((_GRID, 1, _BM), jnp.float32),
        jax.ShapeDtypeStruct((_GRID, 1, _BM), jnp.float32),
        jax.ShapeDtypeStruct((_GRID, 1, _BM), jnp.float32),
    ],
)


@functools.cache
def _build_sc_call():
    mesh = plsc.VectorSubcoreMesh(core_axis_name="c", subcore_axis_name="s",
                                  num_cores=_NC, num_subcores=_NS)
    return functools.partial(
        pl.kernel,
        out_type=[
            jax.ShapeDtypeStruct((_N_E, _E_DIM), jnp.float32),
            jax.ShapeDtypeStruct((2 * _N_E,), jnp.float32),
        ],
        mesh=mesh,
        scratch_types=[
            pltpu.VMEM((_N_E,), jnp.int32),
            pltpu.VMEM((_GCHUNK,), jnp.int32),
            pltpu.VMEM((_GCHUNK, _E_DIM), jnp.float32),
            pltpu.VMEM((_PER_W,), jnp.float32),
            pltpu.SemaphoreType.DMA,
        ],
        compiler_params=pltpu.CompilerParams(needs_layout_passes=False,
                                             use_tc_tiling_on_sc=False),
    )(_sc_body)


def _sc_body(emb_hbm, idx_hbm, zq_hbm, samp_hbm, idx_all, idx_v, rows_v,
             oh_v, sem):
    wid = lax.axis_index("s") * _NC + lax.axis_index("c")
    base = wid * _PER_W

    for k in range(_PER_W // _GCHUNK):
        off = base + k * _GCHUNK
        pltpu.sync_copy(idx_hbm.at[pl.ds(off, _GCHUNK)], idx_v)
        pltpu.async_copy(emb_hbm.at[idx_v], rows_v, sem).wait()
        pltpu.sync_copy(rows_v, zq_hbm.at[pl.ds(off, _GCHUNK)])

    zeros16 = jnp.zeros((16,), jnp.float32)
    for t in range(_PER_W // 16):
        oh_v[pl.ds(t * 16, 16)] = zeros16
    pltpu.sync_copy(oh_v, samp_hbm.at[pl.ds(_N_E + base, _PER_W)])

    pltpu.sync_copy(idx_hbm, idx_all)
    ones16 = jnp.ones((16,), jnp.float32)

    def scan_step(j, carry):
        v = idx_all[pl.ds(j * 16, 16)]
        m = (v >= base) & (v < base + _PER_W)
        loc = jnp.where(m, v - base, 0)
        plsc.store_scatter(oh_v, [loc], ones16, mask=m)
        return carry

    lax.fori_loop(0, _N_E // 16, scan_step, 0, unroll=4)
    pltpu.sync_copy(oh_v, samp_hbm.at[pl.ds(base, _PER_W)])


def kernel(z, emb_weight):
    b, c, h, w, zd = z.shape
    zp = jnp.transpose(z, (0, 2, 3, 4, 1))
    z_flat = zp.reshape(-1, _E_DIM)
    a_z = jnp.sum(z_flat ** 2, axis=1, keepdims=True)
    a_e = jnp.sum(emb_weight ** 2, axis=1)
    aeh_row = (0.5 * a_e).reshape(1, _N_E)

    idx3, mind3, sm3, rs3, rq3 = _dist_call(
        z_flat, a_z, emb_weight, emb_weight, a_e.reshape(1, _N_E), aeh_row)
    idx = idx3.reshape(-1)
    mind = mind3.reshape(-1)

    zq_flat, samp_flat = _build_sc_call()(emb_weight, idx)

    z_q = zq_flat.reshape(zp.shape)
    m = jnp.sum(mind) / jnp.float32(z.size)
    loss = _BETA * m + m
    z_q_st = zp + (z_q - zp)
    z_q_out = jnp.transpose(z_q_st, (0, 4, 1, 2, 3))
    min_enc = idx.reshape(b, h, w, zd)
    sampled_idx = samp_flat.reshape(b, _N_E)
    second_min = a_e + 2.0 * sm3.reshape(-1)
    rs = rs3.reshape(-1)
    rq = rq3.reshape(-1)
    mean_cb_distance = jnp.mean(second_min)
    var_u = (rq - rs * rs / jnp.float32(_N_E)) / jnp.float32(_N_E - 1)
    mean_cb_variance = jnp.mean(4.0 * var_u)
    return z_q_out, loss, (min_enc, sampled_idx, mean_cb_distance,
                           mean_cb_variance)


# speedup vs baseline: 1.0185x; 1.0185x over previous
import functools

import jax
import jax.numpy as jnp
from jax import lax
from jax.experimental import pallas as pl
from jax.experimental.pallas import tpu as pltpu
from jax.experimental.pallas import tpu_sc as plsc

_N_E = 8192
_E_DIM = 32
_BETA = 0.2
_BM = 256
_GRID = _N_E // _BM

_NC = 2
_NS = 16
_NW = _NC * _NS
_PER_W = _N_E // _NW
_GCHUNK = 128


def _dist_body(z_ref, az_ref, embb_ref, emb_ref, ae_ref, aeh_ref,
               idx_ref, mind_ref, sm_ref, rs_ref, rq_ref):
    emb = emb_ref[...]

    zb = z_ref[...]
    az = az_ref[...]
    ae = ae_ref[...]
    p = lax.dot_general(zb, emb, (((1,), (1,)), ((), ())),
                        preferred_element_type=jnp.float32)
    d = (az + ae) - 2.0 * p
    mind = jnp.min(d, axis=1, keepdims=True)
    cols = lax.broadcasted_iota(jnp.int32, d.shape, 1)
    idx = jnp.min(jnp.where(d == mind, cols, _N_E), axis=1)
    idx_ref[0, 0, :] = idx
    mind_ref[0, 0, :] = mind[:, 0]

    i = pl.program_id(0)
    eb = embb_ref[...]
    aeh = aeh_ref[...]
    pc = lax.dot_general(eb, emb, (((1,), (1,)), ((), ())),
                         preferred_element_type=jnp.float32)
    u = aeh - pc
    rows = lax.broadcasted_iota(jnp.int32, u.shape, 0) + i * _BM
    um = jnp.where(rows == cols, jnp.float32(jnp.inf), u)
    sm_ref[0, 0, :] = jnp.min(um, axis=1)
    rs_ref[0, 0, :] = jnp.sum(u, axis=1)
    rq_ref[0, 0, :] = jnp.sum(u * u, axis=1)


_dist_call = pl.pallas_call(
    _dist_body,
    grid=(_GRID,),
    in_specs=[
        pl.BlockSpec((_BM, _E_DIM), lambda i: (i, 0)),
        pl.BlockSpec((_BM, 1), lambda i: (i, 0)),
        pl.BlockSpec((_BM, _E_DIM), lambda i: (i, 0)),
        pl.BlockSpec((_N_E, _E_DIM), lambda i: (0, 0)),
        pl.BlockSpec((1, _N_E), lambda i: (0, 0)),
        pl.BlockSpec((1, _N_E), lambda i: (0, 0)),
    ],
    out_specs=[
        pl.BlockSpec((1, 1, _BM), lambda i: (i, 0, 0)),
        pl.BlockSpec((1, 1, _BM), lambda i: (i, 0, 0)),
        pl.BlockSpec((1, 1, _BM), lambda i: (i, 0, 0)),
        pl.BlockSpec((1, 1, _BM), lambda i: (i, 0, 0)),
        pl.BlockSpec((1, 1, _BM), lambda i: (i, 0, 0)),
    ],
    out_shape=[
        jax.ShapeDtypeStruct((_GRID, 1, _BM), jnp.int32),
        jax.ShapeDtypeStruct((_GRID, 1, _BM), jnp.float32),
        jax.ShapeDtypeStruct((_GRID, 1, _BM), jnp.float32),
        jax.ShapeDtypeStruct((_GRID, 1, _BM), jnp.float32),
        jax.ShapeDtypeStruct((_GRID, 1, _BM), jnp.float32),
    ],
)


@functools.cache
def _build_sc_call():
    mesh = plsc.VectorSubcoreMesh(core_axis_name="c", subcore_axis_name="s",
                                  num_cores=_NC, num_subcores=_NS)
    return functools.partial(
        pl.kernel,
        out_type=[
            jax.ShapeDtypeStruct((_N_E, _E_DIM), jnp.float32),
            jax.ShapeDtypeStruct((2 * _N_E,), jnp.float32),
        ],
        mesh=mesh,
        scratch_types=[
            pltpu.VMEM((_N_E,), jnp.int32),
            pltpu.VMEM((_GCHUNK,), jnp.int32),
            pltpu.VMEM((_GCHUNK, _E_DIM), jnp.float32),
            pltpu.VMEM((_PER_W,), jnp.float32),
            pltpu.SemaphoreType.DMA,
        ],
        compiler_params=pltpu.CompilerParams(needs_layout_passes=False,
                                             use_tc_tiling_on_sc=False),
    )(_sc_body)


def _sc_body(emb_hbm, idx_hbm, zq_hbm, samp_hbm, idx_all, idx_v, rows_v,
             oh_v, sem):
    wid = lax.axis_index("s") * _NC + lax.axis_index("c")
    base = wid * _PER_W

    for k in range(_PER_W // _GCHUNK):
        off = base + k * _GCHUNK
        pltpu.sync_copy(idx_hbm.at[pl.ds(off, _GCHUNK)], idx_v)
        pltpu.async_copy(emb_hbm.at[idx_v], rows_v, sem).wait()
        pltpu.sync_copy(rows_v, zq_hbm.at[pl.ds(off, _GCHUNK)])

    zeros16 = jnp.zeros((16,), jnp.float32)
    for t in range(_PER_W // 16):
        oh_v[pl.ds(t * 16, 16)] = zeros16
    pltpu.sync_copy(oh_v, samp_hbm.at[pl.ds(_N_E + base, _PER_W)])

    pltpu.sync_copy(idx_hbm, idx_all)
    ones16 = jnp.ones((16,), jnp.float32)

    def scan_step(j, carry):
        v = idx_all[pl.ds(j * 16, 16)]
        m = (v >= base) & (v < base + _PER_W)
        loc = jnp.where(m, v - base, 0)
        plsc.store_scatter(oh_v, [loc], ones16, mask=m)
        return carry

    lax.fori_loop(0, _N_E // 16, scan_step, 0, unroll=4)
    pltpu.sync_copy(oh_v, samp_hbm.at[pl.ds(base, _PER_W)])


def kernel(z, emb_weight):
    b, c, h, w, zd = z.shape
    zp = jnp.transpose(z, (0, 2, 3, 4, 1))
    z_flat = zp.reshape(-1, _E_DIM)
    a_z = jnp.sum(z_flat ** 2, axis=1, keepdims=True)
    a_e = jnp.sum(emb_weight ** 2, axis=1)
    aeh_row = (0.5 * a_e).reshape(1, _N_E)

    idx3, mind3, sm3, rs3, rq3 = _dist_call(
        z_flat, a_z, emb_weight, emb_weight, a_e.reshape(1, _N_E), aeh_row)
    idx = idx3.reshape(-1)
    mind = mind3.reshape(-1)

    zq_flat, samp_flat = _build_sc_call()(emb_weight, idx)

    z_q = zq_flat.reshape(zp.shape)
    m = jnp.sum(mind) / jnp.float32(z.size)
    loss = _BETA * m + m
    z_q_st = zp + (z_q - zp)
    z_q_out = jnp.transpose(z_q_st, (0, 4, 1, 2, 3))
    min_enc = idx.reshape(b, h, w, zd)
    sampled_idx = samp_flat.reshape(b, _N_E)
    second_min = a_e + 2.0 * sm3.reshape(-1)
    rs = rs3.reshape(-1)
    rq = rq3.reshape(-1)
    mean_cb_distance = jnp.mean(second_min)
    var_u = (rq - rs * rs / jnp.float32(_N_E)) / jnp.float32(_N_E - 1)
    mean_cb_variance = jnp.mean(4.0 * var_u)
    return z_q_out, loss, (min_enc, sampled_idx, mean_cb_distance,
                           mean_cb_variance)
